# baseline (speedup 1.0000x reference)
import functools

import jax
import jax.numpy as jnp
from jax import lax
from jax.experimental import pallas as pl
from jax.experimental.pallas import tpu as pltpu
from jax.experimental.pallas import tpu_sc as plsc

_EPS = 1e-5


def _adj_counts_sc(adj_t, V):
    E = adj_t.shape[1]
    NC, NS, L = 2, 16, 16
    NW = NC * NS
    epw = E // NW

    mesh = plsc.VectorSubcoreMesh(core_axis_name="c", subcore_axis_name="s")

    @functools.partial(
        pl.kernel,
        mesh=mesh,
        out_type=jax.ShapeDtypeStruct((NC, V * V), jnp.float32),
        scratch_types=[
            pltpu.VMEM((epw,), jnp.int32),
            pltpu.VMEM((epw,), jnp.int32),
            pltpu.VMEM((epw,), jnp.int32),
            pltpu.VMEM((epw,), jnp.float32),
            pltpu.VMEM_SHARED((V * V,), jnp.float32),
        ],
    )
    def adj_k(src_hbm, dst_hbm, zero_hbm, out_hbm, src_v, dst_v, flat_v,
              ones_v, acc_sh):
        c = lax.axis_index("c")
        s = lax.axis_index("s")
        wid = s * NC + c
        base = wid * epw

        @pl.when(s == 0)
        def _init():
            pltpu.sync_copy(zero_hbm, acc_sh)

        plsc.subcore_barrier()

        pltpu.sync_copy(src_hbm.at[pl.ds(base, epw)], src_v)
        pltpu.sync_copy(dst_hbm.at[pl.ds(base, epw)], dst_v)
        for j in range(epw // L):
            sl = pl.ds(j * L, L)
            flat_v[sl] = dst_v[sl] * V + src_v[sl]
            ones_v[sl] = jnp.ones((L,), jnp.float32)
        pltpu.sync_copy(ones_v, acc_sh.at[flat_v], add=True)

        plsc.subcore_barrier()

        @pl.when(s == 0)
        def _dump():
            pltpu.sync_copy(acc_sh, out_hbm.at[c])

    zeros = jnp.zeros((V * V,), jnp.float32)
    out = adj_k(adj_t[0], adj_t[1], zeros)
    return out.reshape(NC, V, V)


def _fused_body(ap, x_ref, wb1, bl1, wb2, bl2, wb3, bl3,
                out_ref, h_s, m2_s, s1, q1, s2, q2, *, bb, b_total):
    p = pl.program_id(0)
    i = pl.program_id(1)
    v = ap.shape[-1]

    @pl.when((p == 0) & (i == 0))
    def _init():
        A = ap[0] + ap[1]
        cnt = jnp.maximum(jnp.sum(A, axis=1, keepdims=True), 1.0)
        Mn = A / cnt
        Z = jnp.zeros_like(Mn)
        m2_s[...] = jnp.concatenate(
            [jnp.concatenate([Mn, Z], 1), jnp.concatenate([Z, Mn], 1)], 0)
        s1[...] = jnp.zeros_like(s1)
        q1[...] = jnp.zeros_like(q1)
        s2[...] = jnp.zeros_like(s2)
        q2[...] = jnp.zeros_like(q2)

    def conv(h, wb, bl):
        c = h.shape[-1]
        hh = wb.shape[0] // 2
        dn = (((1,), (1,)), ((), ()))
        P = lax.dot_general(h.reshape(bb * v, c), wb[...], dn,
                            preferred_element_type=jnp.float32)
        pl2 = P[:, :hh].reshape(bb // 2, 2 * v, hh)
        M2 = m2_s[...]
        mean = jnp.concatenate(
            [jnp.dot(M2, pl2[g], preferred_element_type=jnp.float32)
             for g in range(bb // 2)], axis=0)
        y = mean + P[:, hh:] + bl[...]
        return y.reshape(bb, v, -1)

    def bnrelu(y, s, q):
        mu = s[...] * (1.0 / b_total)
        var = q[...] * (1.0 / b_total) - mu * mu
        inv = lax.rsqrt(var + _EPS)
        return jnp.maximum((y - mu[None]) * inv[None], 0.0)

    sl = pl.ds(i * bb, bb)

    @pl.when(p == 0)
    def _phase0():
        y = conv(x_ref[...], wb1, bl1)
        h_s[sl] = y
        s1[...] += jnp.sum(y, axis=0)
        q1[...] += jnp.sum(y * y, axis=0)

    @pl.when(p == 1)
    def _phase1():
        h = bnrelu(h_s[sl], s1, q1)
        y = conv(h, wb2, bl2)
        h_s[sl] = y
        s2[...] += jnp.sum(y, axis=0)
        q2[...] += jnp.sum(y * y, axis=0)

    @pl.when(p == 2)
    def _phase2():
        h = bnrelu(h_s[sl], s2, q2)
        out_ref[...] = conv(h, wb3, bl3)


def _fused_call(a_parts, xr, weights, bb=64):
    B, V, C = xr.shape
    (wl1, bl1, wr1, wl2, bl2, wr2, wl3, bl3, wr3) = weights
    H = wl3.shape[0]
    grid = (3, B // bb)

    wb1 = jnp.concatenate([wl1, wr1], axis=0)
    wb2 = jnp.concatenate([wl2, wr2], axis=0)
    wb3 = jnp.concatenate([wl3, wr3], axis=0)

    wspec = pl.BlockSpec((2 * H, C), lambda p, i: (0, 0))
    bspec = pl.BlockSpec((1, V), lambda p, i: (0, 0))
    in_specs = [
        pl.BlockSpec((2, V, V), lambda p, i: (0, 0, 0)),
        pl.BlockSpec((bb, V, C), lambda p, i: ((p == 0) * i, 0, 0)),
        wspec, bspec, wspec, bspec, wspec, bspec,
    ]
    args = [a_parts, xr,
            wb1, bl1.reshape(1, -1),
            wb2, bl2.reshape(1, -1),
            wb3, bl3.reshape(1, -1)]

    body = functools.partial(_fused_body, bb=bb, b_total=float(B))
    return pl.pallas_call(
        body,
        grid=grid,
        in_specs=in_specs,
        out_specs=pl.BlockSpec((bb, V, H), lambda p, i: ((p == 2) * i, 0, 0)),
        out_shape=jax.ShapeDtypeStruct((B, V, H), jnp.float32),
        scratch_shapes=[
            pltpu.VMEM((B, V, C), jnp.float32),
            pltpu.VMEM((2 * V, 2 * V), jnp.float32),
            pltpu.VMEM((V, C), jnp.float32),
            pltpu.VMEM((V, C), jnp.float32),
            pltpu.VMEM((V, C), jnp.float32),
            pltpu.VMEM((V, C), jnp.float32),
        ],
    )(*args)


def kernel(x, adj_t, W_l1, b_l1, W_r1, W_l2, b_l2, W_r2, W_l3, b_l3, W_r3):
    N, T, V, C = x.shape
    B = N * T
    xr = x.reshape(B, V, C)

    a_parts = _adj_counts_sc(adj_t, V)
    y = _fused_call(a_parts, xr,
                    (W_l1, b_l1, W_r1, W_l2, b_l2, W_r2, W_l3, b_l3, W_r3))
    return y.reshape(N, T, V, -1)


# speedup vs baseline: 1.0233x; 1.0233x over previous
import functools

import jax
import jax.numpy as jnp
from jax import lax
from jax.experimental import pallas as pl
from jax.experimental.pallas import tpu as pltpu
from jax.experimental.pallas import tpu_sc as plsc

_EPS = 1e-5


def _adj_counts_sc(adj_t, V):
    E = adj_t.shape[1]
    NC, NS, L = 2, 16, 16
    NW = NC * NS
    epw = E // NW

    mesh = plsc.VectorSubcoreMesh(core_axis_name="c", subcore_axis_name="s")

    @functools.partial(
        pl.kernel,
        mesh=mesh,
        out_type=jax.ShapeDtypeStruct((NC, V * V), jnp.float32),
        scratch_types=[
            pltpu.VMEM((epw,), jnp.int32),
            pltpu.VMEM((epw,), jnp.int32),
            pltpu.VMEM((epw,), jnp.int32),
            pltpu.VMEM((epw,), jnp.float32),
            pltpu.VMEM_SHARED((V * V,), jnp.float32),
        ],
    )
    def adj_k(src_hbm, dst_hbm, zero_hbm, out_hbm, src_v, dst_v, flat_v,
              ones_v, acc_sh):
        c = lax.axis_index("c")
        s = lax.axis_index("s")
        wid = s * NC + c
        base = wid * epw

        @pl.when(s == 0)
        def _init():
            pltpu.sync_copy(zero_hbm, acc_sh)

        plsc.subcore_barrier()

        pltpu.sync_copy(src_hbm.at[pl.ds(base, epw)], src_v)
        pltpu.sync_copy(dst_hbm.at[pl.ds(base, epw)], dst_v)
        for j in range(epw // L):
            sl = pl.ds(j * L, L)
            flat_v[sl] = dst_v[sl] * V + src_v[sl]
            ones_v[sl] = jnp.ones((L,), jnp.float32)
        pltpu.sync_copy(ones_v, acc_sh.at[flat_v], add=True)

        plsc.subcore_barrier()

        @pl.when(s == 0)
        def _dump():
            pltpu.sync_copy(acc_sh, out_hbm.at[c])

    zeros = jnp.zeros((V * V,), jnp.float32)
    out = adj_k(adj_t[0], adj_t[1], zeros)
    return out.reshape(NC, V, V)


def _fused_body(ap, x_ref, wl1, bl1, wr1, wl2, bl2, wr2, wl3, bl3, wr3,
                out_ref, h_s, m2_s, s1, q1, s2, q2, *, bb, b_total):
    p = pl.program_id(0)
    i = pl.program_id(1)
    v = ap.shape[-1]

    @pl.when((p == 0) & (i == 0))
    def _init():
        A = ap[0] + ap[1]
        cnt = jnp.maximum(jnp.sum(A, axis=1, keepdims=True), 1.0)
        Mn = A / cnt
        Z = jnp.zeros_like(Mn)
        m2_s[...] = jnp.concatenate(
            [jnp.concatenate([Mn, Z], 1), jnp.concatenate([Z, Mn], 1)], 0)
        s1[...] = jnp.zeros_like(s1)
        q1[...] = jnp.zeros_like(q1)
        s2[...] = jnp.zeros_like(s2)
        q2[...] = jnp.zeros_like(q2)

    def conv(h, wl, bl, wr):
        c = h.shape[-1]
        M2 = m2_s[...]
        h2 = h.reshape(bb // 2, 2 * v, c)
        mean = jnp.concatenate(
            [jnp.dot(M2, h2[g], preferred_element_type=jnp.float32)
             for g in range(bb // 2)], axis=0)
        dn = (((1,), (1,)), ((), ()))
        y = (lax.dot_general(mean, wl[...], dn,
                             preferred_element_type=jnp.float32)
             + lax.dot_general(h.reshape(bb * v, c), wr[...], dn,
                               preferred_element_type=jnp.float32)
             + bl[...])
        return y.reshape(bb, v, -1)

    def bnrelu(y, s, q):
        mu = s[...] * (1.0 / b_total)
        var = q[...] * (1.0 / b_total) - mu * mu
        inv = lax.rsqrt(var + _EPS)
        return jnp.maximum((y - mu[None]) * inv[None], 0.0)

    sl = pl.ds(i * bb, bb)

    @pl.when(p == 0)
    def _phase0():
        y = conv(x_ref[...], wl1, bl1, wr1)
        h_s[sl] = y.astype(h_s.dtype)
        s1[...] += jnp.sum(y, axis=0)
        q1[...] += jnp.sum(y * y, axis=0)

    @pl.when(p == 1)
    def _phase1():
        h = bnrelu(h_s[sl].astype(jnp.float32), s1, q1)
        y = conv(h, wl2, bl2, wr2)
        h_s[sl] = y.astype(h_s.dtype)
        s2[...] += jnp.sum(y, axis=0)
        q2[...] += jnp.sum(y * y, axis=0)

    @pl.when(p == 2)
    def _phase2():
        h = bnrelu(h_s[sl].astype(jnp.float32), s2, q2)
        out_ref[...] = conv(h, wl3, bl3, wr3)


def _fused_call(a_parts, xr, weights, bb=64):
    B, V, C = xr.shape
    (wl1, bl1, wr1, wl2, bl2, wr2, wl3, bl3, wr3) = weights
    H = wl3.shape[0]
    grid = (3, B // bb)

    wspec = pl.BlockSpec((V, V), lambda p, i: (0, 0))
    bspec = pl.BlockSpec((1, V), lambda p, i: (0, 0))
    in_specs = [
        pl.BlockSpec((2, V, V), lambda p, i: (0, 0, 0)),
        pl.BlockSpec((bb, V, C), lambda p, i: ((p == 0) * i, 0, 0)),
        wspec, bspec, wspec, wspec, bspec, wspec, wspec, bspec, wspec,
    ]
    args = [a_parts, xr,
            wl1, bl1.reshape(1, -1), wr1,
            wl2, bl2.reshape(1, -1), wr2,
            wl3, bl3.reshape(1, -1), wr3]

    body = functools.partial(_fused_body, bb=bb, b_total=float(B))
    return pl.pallas_call(
        body,
        grid=grid,
        in_specs=in_specs,
        out_specs=pl.BlockSpec((bb, V, H), lambda p, i: ((p == 2) * i, 0, 0)),
        out_shape=jax.ShapeDtypeStruct((B, V, H), jnp.float32),
        scratch_shapes=[
            pltpu.VMEM((B, V, C), jnp.bfloat16),
            pltpu.VMEM((2 * V, 2 * V), jnp.float32),
            pltpu.VMEM((V, C), jnp.float32),
            pltpu.VMEM((V, C), jnp.float32),
            pltpu.VMEM((V, C), jnp.float32),
            pltpu.VMEM((V, C), jnp.float32),
        ],
    )(*args)


def kernel(x, adj_t, W_l1, b_l1, W_r1, W_l2, b_l2, W_r2, W_l3, b_l3, W_r3):
    N, T, V, C = x.shape
    B = N * T
    xr = x.reshape(B, V, C)

    a_parts = _adj_counts_sc(adj_t, V)
    y = _fused_call(a_parts, xr,
                    (W_l1, b_l1, W_r1, W_l2, b_l2, W_r2, W_l3, b_l3, W_r3))
    return y.reshape(N, T, V, -1)


# speedup vs baseline: 1.0276x; 1.0042x over previous
import functools

import jax
import jax.numpy as jnp
from jax import lax
from jax.experimental import pallas as pl
from jax.experimental.pallas import tpu as pltpu
from jax.experimental.pallas import tpu_sc as plsc

_EPS = 1e-5


def _adj_counts_sc(adj_t, V):
    E = adj_t.shape[1]
    NC, NS, L = 2, 16, 16
    NW = NC * NS
    epw = E // NW

    mesh = plsc.VectorSubcoreMesh(core_axis_name="c", subcore_axis_name="s")

    @functools.partial(
        pl.kernel,
        mesh=mesh,
        out_type=jax.ShapeDtypeStruct((NC, V * V), jnp.float32),
        scratch_types=[
            pltpu.VMEM((epw,), jnp.int32),
            pltpu.VMEM((epw,), jnp.int32),
            pltpu.VMEM((epw,), jnp.int32),
            pltpu.VMEM((epw,), jnp.float32),
            pltpu.VMEM_SHARED((V * V,), jnp.float32),
        ],
    )
    def adj_k(src_hbm, dst_hbm, zero_hbm, out_hbm, src_v, dst_v, flat_v,
              ones_v, acc_sh):
        c = lax.axis_index("c")
        s = lax.axis_index("s")
        wid = s * NC + c
        base = wid * epw

        @pl.when(s == 0)
        def _init():
            pltpu.sync_copy(zero_hbm, acc_sh)

        plsc.subcore_barrier()

        pltpu.sync_copy(src_hbm.at[pl.ds(base, epw)], src_v)
        pltpu.sync_copy(dst_hbm.at[pl.ds(base, epw)], dst_v)
        for j in range(epw // L):
            sl = pl.ds(j * L, L)
            flat_v[sl] = dst_v[sl] * V + src_v[sl]
            ones_v[sl] = jnp.ones((L,), jnp.float32)
        pltpu.sync_copy(ones_v, acc_sh.at[flat_v], add=True)

        plsc.subcore_barrier()

        @pl.when(s == 0)
        def _dump():
            pltpu.sync_copy(acc_sh, out_hbm.at[c])

    zeros = jnp.zeros((V * V,), jnp.float32)
    out = adj_k(adj_t[0], adj_t[1], zeros)
    return out.reshape(NC, V, V)


def _fused_body(ap, x_ref, wl1, bl1, wr1, wl2, bl2, wr2, wl3, bl3, wr3,
                out_ref, h_s, m2_s, s1, q1, s2, q2, *, bb, b_total):
    p = pl.program_id(0)
    i = pl.program_id(1)
    v = ap.shape[-1]

    @pl.when((p == 0) & (i == 0))
    def _init():
        A = ap[0] + ap[1]
        cnt = jnp.maximum(jnp.sum(A, axis=1, keepdims=True), 1.0)
        Mn = A / cnt
        Z = jnp.zeros_like(Mn)
        m2_s[...] = jnp.concatenate(
            [jnp.concatenate([Mn, Z], 1), jnp.concatenate([Z, Mn], 1)], 0)
        s1[...] = jnp.zeros_like(s1)
        q1[...] = jnp.zeros_like(q1)
        s2[...] = jnp.zeros_like(s2)
        q2[...] = jnp.zeros_like(q2)

    def conv(h, wl, bl, wr):
        c = h.shape[-1]
        hb = h.astype(jnp.bfloat16)
        M2 = m2_s[...].astype(jnp.bfloat16)
        h2 = hb.reshape(bb // 2, 2 * v, c)
        mean = jnp.concatenate(
            [jnp.dot(M2, h2[g], preferred_element_type=jnp.float32)
             for g in range(bb // 2)], axis=0)
        dn = (((1,), (1,)), ((), ()))
        y = (lax.dot_general(mean.astype(jnp.bfloat16),
                             wl[...].astype(jnp.bfloat16), dn,
                             preferred_element_type=jnp.float32)
             + lax.dot_general(hb.reshape(bb * v, c),
                               wr[...].astype(jnp.bfloat16), dn,
                               preferred_element_type=jnp.float32)
             + bl[...])
        return y.reshape(bb, v, -1)

    def bnrelu(y, s, q):
        mu = s[...] * (1.0 / b_total)
        var = q[...] * (1.0 / b_total) - mu * mu
        inv = lax.rsqrt(var + _EPS)
        return jnp.maximum((y - mu[None]) * inv[None], 0.0)

    sl = pl.ds(i * bb, bb)

    @pl.when(p == 0)
    def _phase0():
        y = conv(x_ref[...], wl1, bl1, wr1)
        h_s[sl] = y.astype(h_s.dtype)
        s1[...] += jnp.sum(y, axis=0)
        q1[...] += jnp.sum(y * y, axis=0)

    @pl.when(p == 1)
    def _phase1():
        h = bnrelu(h_s[sl].astype(jnp.float32), s1, q1)
        y = conv(h, wl2, bl2, wr2)
        h_s[sl] = y.astype(h_s.dtype)
        s2[...] += jnp.sum(y, axis=0)
        q2[...] += jnp.sum(y * y, axis=0)

    @pl.when(p == 2)
    def _phase2():
        h = bnrelu(h_s[sl].astype(jnp.float32), s2, q2)
        out_ref[...] = conv(h, wl3, bl3, wr3)


def _fused_call(a_parts, xr, weights, bb=64):
    B, V, C = xr.shape
    (wl1, bl1, wr1, wl2, bl2, wr2, wl3, bl3, wr3) = weights
    H = wl3.shape[0]
    grid = (3, B // bb)

    wspec = pl.BlockSpec((V, V), lambda p, i: (0, 0))
    bspec = pl.BlockSpec((1, V), lambda p, i: (0, 0))
    in_specs = [
        pl.BlockSpec((2, V, V), lambda p, i: (0, 0, 0)),
        pl.BlockSpec((bb, V, C), lambda p, i: ((p == 0) * i, 0, 0)),
        wspec, bspec, wspec, wspec, bspec, wspec, wspec, bspec, wspec,
    ]
    args = [a_parts, xr,
            wl1, bl1.reshape(1, -1), wr1,
            wl2, bl2.reshape(1, -1), wr2,
            wl3, bl3.reshape(1, -1), wr3]

    body = functools.partial(_fused_body, bb=bb, b_total=float(B))
    return pl.pallas_call(
        body,
        grid=grid,
        in_specs=in_specs,
        out_specs=pl.BlockSpec((bb, V, H), lambda p, i: ((p == 2) * i, 0, 0)),
        out_shape=jax.ShapeDtypeStruct((B, V, H), jnp.float32),
        scratch_shapes=[
            pltpu.VMEM((B, V, C), jnp.float32),
            pltpu.VMEM((2 * V, 2 * V), jnp.float32),
            pltpu.VMEM((V, C), jnp.float32),
            pltpu.VMEM((V, C), jnp.float32),
            pltpu.VMEM((V, C), jnp.float32),
            pltpu.VMEM((V, C), jnp.float32),
        ],
    )(*args)


def kernel(x, adj_t, W_l1, b_l1, W_r1, W_l2, b_l2, W_r2, W_l3, b_l3, W_r3):
    N, T, V, C = x.shape
    B = N * T
    xr = x.reshape(B, V, C)

    a_parts = _adj_counts_sc(adj_t, V)
    y = _fused_call(a_parts, xr,
                    (W_l1, b_l1, W_r1, W_l2, b_l2, W_r2, W_l3, b_l3, W_r3))
    return y.reshape(N, T, V, -1)
